# Initial kernel scaffold; baseline (speedup 1.0000x reference)
import jax
import jax.numpy as jnp
from jax.experimental import pallas as pl


def kernel(x, edge_index, edge_attr, W1, a_src1, a_dst1, b1, W2, a_src2, a_dst2, b2):
    raise NotImplementedError("write your pallas kernel here")


# speedup vs baseline: 36.4896x; 36.4896x over previous
import functools

import jax
import jax.numpy as jnp
from jax import lax
from jax.experimental import pallas as pl
from jax.experimental.pallas import tpu as pltpu
from jax.experimental.pallas import tpu_sc as plsc

N = 10000
NFEAT = 128
HID = 16
HEADS = 8
NCLASS = 16
E = 320000

NW = 32
B = 128
NB = 81
CHUNK = NB * B
E_PAD = NW * CHUNK
NPAD = 10112
ROWS_PER_TILE = NPAD // 16

_NEG = -1.0e30


def _tc1_body(x_ref, w1_ref, acs_ref, acd_ref, h_ref, as_ref, ad_ref):
    h = jnp.dot(x_ref[...], w1_ref[...], preferred_element_type=jnp.float32)
    h_ref[...] = h
    as_ref[...] = jnp.dot(h, acs_ref[...], preferred_element_type=jnp.float32)
    ad_ref[...] = jnp.dot(h, acd_ref[...], preferred_element_type=jnp.float32)


def _tc2_body(na_ref, nb_ref, da_ref, db_ref, s_ref, b1_ref, w2_ref,
              a2s_ref, a2d_ref, h2_ref, as2_ref, ad2_ref):
    den = da_ref[...] + db_ref[...]
    div = jnp.dot(den, s_ref[...], preferred_element_type=jnp.float32)
    out1 = (na_ref[...] + nb_ref[...]) / div + b1_ref[...]
    x2 = jnp.where(out1 > 0, out1, jnp.exp(out1) - 1.0)
    h2 = jnp.dot(x2, w2_ref[...], preferred_element_type=jnp.float32)
    h2_ref[...] = h2
    as2_ref[...] = jnp.dot(h2, a2s_ref[...], preferred_element_type=jnp.float32)
    ad2_ref[...] = jnp.dot(h2, a2d_ref[...], preferred_element_type=jnp.float32)


def _tc3_body(na_ref, nb_ref, da_ref, db_ref, b2_ref, fin_ref, lp_ref):
    fin = (na_ref[...] + nb_ref[...]) / (da_ref[...] + db_ref[...]) + b2_ref[...]
    fin_ref[...] = fin
    m = jnp.max(fin, axis=1, keepdims=True)
    lse = jnp.log(jnp.sum(jnp.exp(fin - m), axis=1, keepdims=True)) + m
    lp_ref[...] = fin - lse


def _make_sc_gat(D, heads):
    mesh = plsc.VectorSubcoreMesh(core_axis_name="c", subcore_axis_name="s")
    chan = D // heads

    @functools.partial(
        pl.kernel,
        out_type=[
            jax.ShapeDtypeStruct((2, NPAD, D), jnp.float32),
            jax.ShapeDtypeStruct((2, NPAD, 16), jnp.float32),
        ],
        mesh=mesh,
        compiler_params=pltpu.CompilerParams(use_tc_tiling_on_sc=False),
        scratch_types=[
            pltpu.VMEM((B,), jnp.int32),
            pltpu.VMEM((B,), jnp.int32),
            pltpu.VMEM((B, D), jnp.float32),
            pltpu.VMEM((B, D), jnp.float32),
            pltpu.VMEM((B, 16), jnp.float32),
            pltpu.VMEM((B, 16), jnp.float32),
            pltpu.VMEM((B, 16), jnp.float32),
            pltpu.VMEM_SHARED((NPAD, D), jnp.float32),
            pltpu.VMEM_SHARED((NPAD, 16), jnp.float32),
            pltpu.SemaphoreType.DMA,
        ],
    )
    def sc_gat(src_hbm, dst_hbm, h_hbm, as_hbm, ad_hbm, num_hbm, den_hbm,
               src_v, dst_v, hs_v, msg_v, as_v, ad_v, w_v, num_s, den_s, sem):
        c = lax.axis_index("c")
        s = lax.axis_index("s")
        wid = c * 16 + s

        def _zrow(i, _):
            zero = jnp.zeros((16,), jnp.float32)
            for j in range(D // 16):
                msg_v[i, pl.ds(j * 16, 16)] = zero
            w_v[i, pl.ds(0, 16)] = zero
            return 0
        lax.fori_loop(0, B, _zrow, 0)
        rb = s * ROWS_PER_TILE
        full, rem = ROWS_PER_TILE // B, ROWS_PER_TILE % B
        for k in range(full):
            pltpu.sync_copy(msg_v, num_s.at[pl.ds(rb + k * B, B)])
            pltpu.sync_copy(w_v, den_s.at[pl.ds(rb + k * B, B)])
        if rem:
            pltpu.sync_copy(msg_v.at[pl.ds(0, rem)],
                            num_s.at[pl.ds(rb + full * B, rem)])
            pltpu.sync_copy(w_v.at[pl.ds(0, rem)],
                            den_s.at[pl.ds(rb + full * B, rem)])
        plsc.subcore_barrier()

        def _batch(b, _):
            base = wid * CHUNK + b * B
            pltpu.sync_copy(src_hbm.at[pl.ds(base, B)], src_v)
            pltpu.sync_copy(dst_hbm.at[pl.ds(base, B)], dst_v)
            pltpu.async_copy(h_hbm.at[src_v], hs_v, sem).wait()
            pltpu.async_copy(as_hbm.at[src_v], as_v, sem).wait()
            pltpu.async_copy(ad_hbm.at[dst_v], ad_v, sem).wait()

            def _edge(i, _):
                e = as_v[i] + ad_v[i]
                e = jnp.where(e >= 0.0, e, 0.2 * e)
                w = jnp.exp(e)
                w_v[i] = w
                if heads == 1:
                    msg_v[i] = hs_v[i] * w
                else:
                    for h in range(heads):
                        wh = jnp.broadcast_to(w[h], (16,))
                        msg_v[i, pl.ds(h * chan, chan)] = (
                            hs_v[i, pl.ds(h * chan, chan)] * wh)
                return 0
            lax.fori_loop(0, B, _edge, 0)

            pltpu.sync_copy(msg_v, num_s.at[dst_v], add=True)
            pltpu.sync_copy(w_v, den_s.at[dst_v], add=True)
            return 0
        lax.fori_loop(0, NB, _batch, 0)

        plsc.subcore_barrier()
        pltpu.sync_copy(num_s.at[pl.ds(rb, ROWS_PER_TILE)],
                        num_hbm.at[c, pl.ds(rb, ROWS_PER_TILE)])
        pltpu.sync_copy(den_s.at[pl.ds(rb, ROWS_PER_TILE)],
                        den_hbm.at[c, pl.ds(rb, ROWS_PER_TILE)])

    return sc_gat


_sc_gat_l1 = _make_sc_gat(HEADS * HID, HEADS)
_sc_gat_l2 = _make_sc_gat(NCLASS, 1)


def _head_matrix(a):
    h, cch = a.shape
    rows = jnp.arange(h * cch)
    cols = rows // cch
    m = jnp.zeros((h * cch, 8), jnp.float32).at[rows, cols].set(a.reshape(-1))
    return jnp.concatenate([m, m], axis=1)


def kernel(x, edge_index, edge_attr, W1, a_src1, a_dst1, b1,
           W2, a_src2, a_dst2, b2):
    acs = _head_matrix(a_src1)
    acd = _head_matrix(a_dst1)
    a2s = jnp.tile(a_src2.reshape(NCLASS, 1), (1, 16))
    a2d = jnp.tile(a_dst2.reshape(NCLASS, 1), (1, 16))
    s_rows = jnp.arange(16)
    s_cols = jnp.arange(HEADS * HID)
    s_mat = (s_rows[:, None] == (s_cols[None, :] // HID)).astype(jnp.float32)

    loop = jnp.arange(N, dtype=jnp.int32)
    pad = jnp.full((E_PAD - E - N,), N, dtype=jnp.int32)
    src_all = jnp.concatenate([edge_index[0], loop, pad])
    dst_all = jnp.concatenate([edge_index[1], loop, pad])

    h1, as1, ad1 = pl.pallas_call(
        _tc1_body,
        out_shape=[
            jax.ShapeDtypeStruct((N, HEADS * HID), jnp.float32),
            jax.ShapeDtypeStruct((N, 16), jnp.float32),
            jax.ShapeDtypeStruct((N, 16), jnp.float32),
        ],
    )(x, W1, acs, acd)

    hpad = ((0, NPAD - N), (0, 0))
    h1e = jnp.pad(h1, hpad)
    as1e = jnp.pad(as1, hpad, constant_values=_NEG)
    ad1e = jnp.pad(ad1, hpad, constant_values=_NEG)

    num1, den1 = _sc_gat_l1(src_all, dst_all, h1e, as1e, ad1e)

    h2, as2, ad2 = pl.pallas_call(
        _tc2_body,
        out_shape=[
            jax.ShapeDtypeStruct((N, NCLASS), jnp.float32),
            jax.ShapeDtypeStruct((N, 16), jnp.float32),
            jax.ShapeDtypeStruct((N, 16), jnp.float32),
        ],
    )(num1[0, :N], num1[1, :N], den1[0, :N], den1[1, :N],
      s_mat, b1.reshape(1, HEADS * HID), W2, a2s, a2d)

    h2e = jnp.pad(h2, hpad)
    as2e = jnp.pad(as2, hpad, constant_values=_NEG)
    ad2e = jnp.pad(ad2, hpad, constant_values=_NEG)

    num2, den2 = _sc_gat_l2(src_all, dst_all, h2e, as2e, ad2e)

    final, logp = pl.pallas_call(
        _tc3_body,
        out_shape=[
            jax.ShapeDtypeStruct((N, NCLASS), jnp.float32),
            jax.ShapeDtypeStruct((N, NCLASS), jnp.float32),
        ],
    )(num2[0, :N], num2[1, :N], den2[0, :N], den2[1, :N],
      b2.reshape(1, NCLASS))

    return (final, logp)


# speedup vs baseline: 41.2394x; 1.1302x over previous
import functools

import jax
import jax.numpy as jnp
from jax import lax
from jax.experimental import pallas as pl
from jax.experimental.pallas import tpu as pltpu
from jax.experimental.pallas import tpu_sc as plsc

N = 10000
NFEAT = 128
HID = 16
HEADS = 8
NCLASS = 16
E = 320000

NW = 32
CHUNK = 10496
E_PAD = NW * CHUNK
NPAD = 10112
ROWS_PER_TILE = NPAD // 16

_NEG = -1.0e30


def _tc1_body(x_ref, w1_ref, acs_ref, acd_ref, h_ref, as_ref, ad_ref):
    h = jnp.dot(x_ref[...], w1_ref[...], preferred_element_type=jnp.float32)
    h_ref[...] = h
    as_ref[...] = jnp.dot(h, acs_ref[...], preferred_element_type=jnp.float32)
    ad_ref[...] = jnp.dot(h, acd_ref[...], preferred_element_type=jnp.float32)


def _tc2_body(na_ref, nb_ref, da_ref, db_ref, s_ref, b1_ref, w2_ref,
              a2s_ref, a2d_ref, h2_ref, as2_ref, ad2_ref):
    den = da_ref[...] + db_ref[...]
    div = jnp.dot(den, s_ref[...], preferred_element_type=jnp.float32)
    out1 = (na_ref[...] + nb_ref[...]) / div + b1_ref[...]
    x2 = jnp.where(out1 > 0, out1, jnp.exp(out1) - 1.0)
    h2 = jnp.dot(x2, w2_ref[...], preferred_element_type=jnp.float32)
    h2_ref[...] = h2
    as2_ref[...] = jnp.dot(h2, a2s_ref[...], preferred_element_type=jnp.float32)
    ad2_ref[...] = jnp.dot(h2, a2d_ref[...], preferred_element_type=jnp.float32)


def _tc3_body(na_ref, nb_ref, da_ref, db_ref, b2_ref, fin_ref, lp_ref):
    fin = (na_ref[...] + nb_ref[...]) / (da_ref[...] + db_ref[...]) + b2_ref[...]
    fin_ref[...] = fin
    m = jnp.max(fin, axis=1, keepdims=True)
    lse = jnp.log(jnp.sum(jnp.exp(fin - m), axis=1, keepdims=True)) + m
    lp_ref[...] = fin - lse


def _make_sc_gat(D, heads, B):
    mesh = plsc.VectorSubcoreMesh(core_axis_name="c", subcore_axis_name="s")
    chan = D // heads
    nb = CHUNK // B
    assert nb * B == CHUNK and nb % 2 == 0
    pairs = nb // 2

    @functools.partial(
        pl.kernel,
        out_type=[
            jax.ShapeDtypeStruct((2, NPAD, D), jnp.float32),
            jax.ShapeDtypeStruct((2, NPAD, 16), jnp.float32),
        ],
        mesh=mesh,
        compiler_params=pltpu.CompilerParams(use_tc_tiling_on_sc=False),
        scratch_types=[
            pltpu.VMEM((B,), jnp.int32),
            pltpu.VMEM((B,), jnp.int32),
            pltpu.VMEM((B, D), jnp.float32),
            pltpu.VMEM((B, D), jnp.float32),
            pltpu.VMEM((B, 16), jnp.float32),
            pltpu.VMEM((B, 16), jnp.float32),
            pltpu.VMEM((B, 16), jnp.float32),
            pltpu.VMEM((B,), jnp.int32),
            pltpu.VMEM((B,), jnp.int32),
            pltpu.VMEM((B, D), jnp.float32),
            pltpu.VMEM((B, D), jnp.float32),
            pltpu.VMEM((B, 16), jnp.float32),
            pltpu.VMEM((B, 16), jnp.float32),
            pltpu.VMEM((B, 16), jnp.float32),
            pltpu.VMEM_SHARED((NPAD, D), jnp.float32),
            pltpu.VMEM_SHARED((NPAD, 16), jnp.float32),
            pltpu.SemaphoreType.DMA,
            pltpu.SemaphoreType.DMA,
            pltpu.SemaphoreType.DMA,
            pltpu.SemaphoreType.DMA,
        ],
    )
    def sc_gat(src_hbm, dst_hbm, h_hbm, as_hbm, ad_hbm, num_hbm, den_hbm,
               src0, dst0, hs0, msg0, as0, ad0, w0,
               src1, dst1, hs1, msg1, as1, ad1, w1,
               num_s, den_s, sg0, sg1, ss0, ss1):
        c = lax.axis_index("c")
        s = lax.axis_index("s")
        wid = c * 16 + s
        slots = ((src0, dst0, hs0, msg0, as0, ad0, w0, sg0, ss0),
                 (src1, dst1, hs1, msg1, as1, ad1, w1, sg1, ss1))

        def prime(S, b):
            sv, dv, hs, msg, asv, adv, wv, sg, ss = S
            base = wid * CHUNK + b * B
            pltpu.sync_copy(src_hbm.at[pl.ds(base, B)], sv)
            pltpu.sync_copy(dst_hbm.at[pl.ds(base, B)], dv)
            pltpu.async_copy(h_hbm.at[sv], hs, sg)
            pltpu.async_copy(as_hbm.at[sv], asv, sg)
            pltpu.async_copy(ad_hbm.at[dv], adv, sg)

        def wait_gathers(S):
            sv, dv, hs, msg, asv, adv, wv, sg, ss = S
            pltpu.make_async_copy(h_hbm.at[sv], hs, sg).wait()
            pltpu.make_async_copy(as_hbm.at[sv], asv, sg).wait()
            pltpu.make_async_copy(ad_hbm.at[dv], adv, sg).wait()

        def scatter(S):
            sv, dv, hs, msg, asv, adv, wv, sg, ss = S
            pltpu.async_copy(msg, num_s.at[dv], ss, add=True)
            pltpu.async_copy(wv, den_s.at[dv], ss, add=True)

        def wait_scatter(S):
            sv, dv, hs, msg, asv, adv, wv, sg, ss = S
            pltpu.make_async_copy(msg, num_s.at[dv], ss).wait()
            pltpu.make_async_copy(wv, den_s.at[dv], ss).wait()

        def compute(S):
            sv, dv, hs, msg, asv, adv, wv, sg, ss = S

            def _edge(i, _):
                e = asv[i] + adv[i]
                e = jnp.where(e >= 0.0, e, 0.2 * e)
                w = jnp.exp(e)
                wv[i] = w
                if heads == 1:
                    msg[i] = hs[i] * w
                else:
                    for h in range(heads):
                        wh = jnp.broadcast_to(w[h], (16,))
                        msg[i, pl.ds(h * chan, chan)] = (
                            hs[i, pl.ds(h * chan, chan)] * wh)
                return 0
            lax.fori_loop(0, B, _edge, 0, unroll=2)

        def _zrow(i, _):
            zero = jnp.zeros((16,), jnp.float32)
            for j in range(D // 16):
                msg0[i, pl.ds(j * 16, 16)] = zero
            w0[i, pl.ds(0, 16)] = zero
            return 0
        lax.fori_loop(0, B, _zrow, 0)
        rb = s * ROWS_PER_TILE
        full, rem = ROWS_PER_TILE // B, ROWS_PER_TILE % B
        for k in range(full):
            pltpu.sync_copy(msg0, num_s.at[pl.ds(rb + k * B, B)])
            pltpu.sync_copy(w0, den_s.at[pl.ds(rb + k * B, B)])
        if rem:
            pltpu.sync_copy(msg0.at[pl.ds(0, rem)],
                            num_s.at[pl.ds(rb + full * B, rem)])
            pltpu.sync_copy(w0.at[pl.ds(0, rem)],
                            den_s.at[pl.ds(rb + full * B, rem)])
        prime(slots[0], 0)
        prime(slots[1], 1)
        plsc.subcore_barrier()

        def _pair(p, _):
            b0 = 2 * p
            wait_gathers(slots[0])
            compute(slots[0])
            scatter(slots[0])
            wait_gathers(slots[1])
            compute(slots[1])
            scatter(slots[1])

            @pl.when(p < pairs - 1)
            def _():
                wait_scatter(slots[0])
                prime(slots[0], b0 + 2)
                wait_scatter(slots[1])
                prime(slots[1], b0 + 3)
            return 0
        lax.fori_loop(0, pairs, _pair, 0)
        wait_scatter(slots[0])
        wait_scatter(slots[1])

        plsc.subcore_barrier()
        pltpu.sync_copy(num_s.at[pl.ds(rb, ROWS_PER_TILE)],
                        num_hbm.at[c, pl.ds(rb, ROWS_PER_TILE)])
        pltpu.sync_copy(den_s.at[pl.ds(rb, ROWS_PER_TILE)],
                        den_hbm.at[c, pl.ds(rb, ROWS_PER_TILE)])

    return sc_gat


_sc_gat_l1 = _make_sc_gat(HEADS * HID, HEADS, 64)
_sc_gat_l2 = _make_sc_gat(NCLASS, 1, 128)


def _head_matrix(a):
    h, cch = a.shape
    rows = jnp.arange(h * cch)
    cols = rows // cch
    m = jnp.zeros((h * cch, 8), jnp.float32).at[rows, cols].set(a.reshape(-1))
    return jnp.concatenate([m, m], axis=1)


def kernel(x, edge_index, edge_attr, W1, a_src1, a_dst1, b1,
           W2, a_src2, a_dst2, b2):
    acs = _head_matrix(a_src1)
    acd = _head_matrix(a_dst1)
    a2s = jnp.tile(a_src2.reshape(NCLASS, 1), (1, 16))
    a2d = jnp.tile(a_dst2.reshape(NCLASS, 1), (1, 16))
    s_rows = jnp.arange(16)
    s_cols = jnp.arange(HEADS * HID)
    s_mat = (s_rows[:, None] == (s_cols[None, :] // HID)).astype(jnp.float32)

    loop = jnp.arange(N, dtype=jnp.int32)
    pad = jnp.full((E_PAD - E - N,), N, dtype=jnp.int32)
    src_all = jnp.concatenate([edge_index[0], loop, pad])
    dst_all = jnp.concatenate([edge_index[1], loop, pad])

    h1, as1, ad1 = pl.pallas_call(
        _tc1_body,
        out_shape=[
            jax.ShapeDtypeStruct((N, HEADS * HID), jnp.float32),
            jax.ShapeDtypeStruct((N, 16), jnp.float32),
            jax.ShapeDtypeStruct((N, 16), jnp.float32),
        ],
    )(x, W1, acs, acd)

    hpad = ((0, NPAD - N), (0, 0))
    h1e = jnp.pad(h1, hpad)
    as1e = jnp.pad(as1, hpad, constant_values=_NEG)
    ad1e = jnp.pad(ad1, hpad, constant_values=_NEG)

    num1, den1 = _sc_gat_l1(src_all, dst_all, h1e, as1e, ad1e)

    h2, as2, ad2 = pl.pallas_call(
        _tc2_body,
        out_shape=[
            jax.ShapeDtypeStruct((N, NCLASS), jnp.float32),
            jax.ShapeDtypeStruct((N, 16), jnp.float32),
            jax.ShapeDtypeStruct((N, 16), jnp.float32),
        ],
    )(num1[0, :N], num1[1, :N], den1[0, :N], den1[1, :N],
      s_mat, b1.reshape(1, HEADS * HID), W2, a2s, a2d)

    h2e = jnp.pad(h2, hpad)
    as2e = jnp.pad(as2, hpad, constant_values=_NEG)
    ad2e = jnp.pad(ad2, hpad, constant_values=_NEG)

    num2, den2 = _sc_gat_l2(src_all, dst_all, h2e, as2e, ad2e)

    final, logp = pl.pallas_call(
        _tc3_body,
        out_shape=[
            jax.ShapeDtypeStruct((N, NCLASS), jnp.float32),
            jax.ShapeDtypeStruct((N, NCLASS), jnp.float32),
        ],
    )(num2[0, :N], num2[1, :N], den2[0, :N], den2[1, :N],
      b2.reshape(1, NCLASS))

    return (final, logp)


# speedup vs baseline: 63.2076x; 1.5327x over previous
import functools

import jax
import jax.numpy as jnp
from jax import lax
from jax.experimental import pallas as pl
from jax.experimental.pallas import tpu as pltpu
from jax.experimental.pallas import tpu_sc as plsc

N = 10000
NFEAT = 128
HID = 16
HEADS = 8
NCLASS = 16
E = 320000

NW = 32
CHUNK = 10496
E_PAD = NW * CHUNK
NPAD = 10112
ROWS_PER_TILE = NPAD // 16

_NEG = -1.0e30

_GATHER_DN = lax.GatherDimensionNumbers(
    offset_dims=(), collapsed_slice_dims=(0,), start_index_map=(0,))


def _lane_bcast(v, h):
    idx = jnp.full((16, 1), h, dtype=jnp.int32)
    return lax.gather(v, idx, _GATHER_DN, (1,),
                      mode=lax.GatherScatterMode.PROMISE_IN_BOUNDS)


def _tc1_body(x_ref, w1_ref, acs_ref, acd_ref, h_ref, as_ref, ad_ref):
    h = jnp.dot(x_ref[...], w1_ref[...], preferred_element_type=jnp.float32)
    h_ref[...] = h
    as_ref[...] = jnp.dot(h, acs_ref[...], preferred_element_type=jnp.float32)
    ad_ref[...] = jnp.dot(h, acd_ref[...], preferred_element_type=jnp.float32)


def _tc2_body(na_ref, nb_ref, da_ref, db_ref, s_ref, b1_ref, w2_ref,
              a2s_ref, a2d_ref, h2_ref, as2_ref, ad2_ref):
    den = da_ref[...] + db_ref[...]
    div = jnp.dot(den, s_ref[...], preferred_element_type=jnp.float32)
    out1 = (na_ref[...] + nb_ref[...]) / div + b1_ref[...]
    x2 = jnp.where(out1 > 0, out1, jnp.exp(out1) - 1.0)
    h2 = jnp.dot(x2, w2_ref[...], preferred_element_type=jnp.float32)
    h2_ref[...] = h2
    as2_ref[...] = jnp.dot(h2, a2s_ref[...], preferred_element_type=jnp.float32)
    ad2_ref[...] = jnp.dot(h2, a2d_ref[...], preferred_element_type=jnp.float32)


def _tc3_body(na_ref, nb_ref, da_ref, db_ref, b2_ref, fin_ref, lp_ref):
    fin = (na_ref[...] + nb_ref[...]) / (da_ref[...] + db_ref[...]) + b2_ref[...]
    fin_ref[...] = fin
    m = jnp.max(fin, axis=1, keepdims=True)
    lse = jnp.log(jnp.sum(jnp.exp(fin - m), axis=1, keepdims=True)) + m
    lp_ref[...] = fin - lse


def _make_sc_gat(D, heads, B):
    mesh = plsc.VectorSubcoreMesh(core_axis_name="c", subcore_axis_name="s")
    chan = D // heads
    nb = CHUNK // B
    assert nb * B == CHUNK and nb % 2 == 0
    pairs = nb // 2

    @functools.partial(
        pl.kernel,
        out_type=[
            jax.ShapeDtypeStruct((2, NPAD, D), jnp.float32),
            jax.ShapeDtypeStruct((2, NPAD, 16), jnp.float32),
        ],
        mesh=mesh,
        compiler_params=pltpu.CompilerParams(use_tc_tiling_on_sc=False),
        scratch_types=[
            pltpu.VMEM((B,), jnp.int32),
            pltpu.VMEM((B,), jnp.int32),
            pltpu.VMEM((B, D), jnp.float32),
            pltpu.VMEM((B, D), jnp.float32),
            pltpu.VMEM((B, 16), jnp.float32),
            pltpu.VMEM((B, 16), jnp.float32),
            pltpu.VMEM((B, 16), jnp.float32),
            pltpu.VMEM((B,), jnp.int32),
            pltpu.VMEM((B,), jnp.int32),
            pltpu.VMEM((B, D), jnp.float32),
            pltpu.VMEM((B, D), jnp.float32),
            pltpu.VMEM((B, 16), jnp.float32),
            pltpu.VMEM((B, 16), jnp.float32),
            pltpu.VMEM((B, 16), jnp.float32),
            pltpu.VMEM_SHARED((NPAD, D), jnp.float32),
            pltpu.VMEM_SHARED((NPAD, 16), jnp.float32),
            pltpu.SemaphoreType.DMA,
            pltpu.SemaphoreType.DMA,
            pltpu.SemaphoreType.DMA,
            pltpu.SemaphoreType.DMA,
        ],
    )
    def sc_gat(src_hbm, dst_hbm, h_hbm, as_hbm, ad_hbm, num_hbm, den_hbm,
               src0, dst0, hs0, msg0, as0, ad0, w0,
               src1, dst1, hs1, msg1, as1, ad1, w1,
               num_s, den_s, sg0, sg1, ss0, ss1):
        c = lax.axis_index("c")
        s = lax.axis_index("s")
        wid = c * 16 + s
        slots = ((src0, dst0, hs0, msg0, as0, ad0, w0, sg0, ss0),
                 (src1, dst1, hs1, msg1, as1, ad1, w1, sg1, ss1))

        def prime(S, b):
            sv, dv, hs, msg, asv, adv, wv, sg, ss = S
            base = wid * CHUNK + b * B
            pltpu.sync_copy(src_hbm.at[pl.ds(base, B)], sv)
            pltpu.sync_copy(dst_hbm.at[pl.ds(base, B)], dv)
            pltpu.async_copy(h_hbm.at[sv], hs, sg)
            pltpu.async_copy(as_hbm.at[sv], asv, sg)
            pltpu.async_copy(ad_hbm.at[dv], adv, sg)

        def wait_gathers(S):
            sv, dv, hs, msg, asv, adv, wv, sg, ss = S
            pltpu.make_async_copy(h_hbm.at[sv], hs, sg).wait()
            pltpu.make_async_copy(as_hbm.at[sv], asv, sg).wait()
            pltpu.make_async_copy(ad_hbm.at[dv], adv, sg).wait()

        def scatter(S):
            sv, dv, hs, msg, asv, adv, wv, sg, ss = S
            pltpu.async_copy(msg, num_s.at[dv], ss, add=True)
            pltpu.async_copy(wv, den_s.at[dv], ss, add=True)

        def wait_scatter(S):
            sv, dv, hs, msg, asv, adv, wv, sg, ss = S
            pltpu.make_async_copy(msg, num_s.at[dv], ss).wait()
            pltpu.make_async_copy(wv, den_s.at[dv], ss).wait()

        def compute(S):
            sv, dv, hs, msg, asv, adv, wv, sg, ss = S

            @plsc.parallel_loop(0, B, 1, unroll=4)
            def _edge(i):
                e = asv[i] + adv[i]
                e = jnp.maximum(e, 0.2 * e)
                w = jnp.exp(e)
                wv[i] = w
                if heads == 1:
                    msg[i] = hs[i] * w
                else:
                    for h in range(heads):
                        wh = _lane_bcast(w, h)
                        msg[i, pl.ds(h * chan, chan)] = (
                            hs[i, pl.ds(h * chan, chan)] * wh)

        def _zrow(i, _):
            zero = jnp.zeros((16,), jnp.float32)
            for j in range(D // 16):
                msg0[i, pl.ds(j * 16, 16)] = zero
            w0[i, pl.ds(0, 16)] = zero
            return 0
        lax.fori_loop(0, B, _zrow, 0)
        rb = s * ROWS_PER_TILE
        full, rem = ROWS_PER_TILE // B, ROWS_PER_TILE % B
        for k in range(full):
            pltpu.sync_copy(msg0, num_s.at[pl.ds(rb + k * B, B)])
            pltpu.sync_copy(w0, den_s.at[pl.ds(rb + k * B, B)])
        if rem:
            pltpu.sync_copy(msg0.at[pl.ds(0, rem)],
                            num_s.at[pl.ds(rb + full * B, rem)])
            pltpu.sync_copy(w0.at[pl.ds(0, rem)],
                            den_s.at[pl.ds(rb + full * B, rem)])
        prime(slots[0], 0)
        prime(slots[1], 1)
        plsc.subcore_barrier()

        def _pair(p, _):
            b0 = 2 * p
            wait_gathers(slots[0])
            compute(slots[0])
            scatter(slots[0])
            wait_gathers(slots[1])
            compute(slots[1])
            scatter(slots[1])

            @pl.when(p < pairs - 1)
            def _():
                wait_scatter(slots[0])
                prime(slots[0], b0 + 2)
                wait_scatter(slots[1])
                prime(slots[1], b0 + 3)
            return 0
        lax.fori_loop(0, pairs, _pair, 0)
        wait_scatter(slots[0])
        wait_scatter(slots[1])

        plsc.subcore_barrier()
        pltpu.sync_copy(num_s.at[pl.ds(rb, ROWS_PER_TILE)],
                        num_hbm.at[c, pl.ds(rb, ROWS_PER_TILE)])
        pltpu.sync_copy(den_s.at[pl.ds(rb, ROWS_PER_TILE)],
                        den_hbm.at[c, pl.ds(rb, ROWS_PER_TILE)])

    return sc_gat


_sc_gat_l1 = _make_sc_gat(HEADS * HID, HEADS, 64)
_sc_gat_l2 = _make_sc_gat(NCLASS, 1, 128)


def _head_matrix(a):
    h, cch = a.shape
    rows = jnp.arange(h * cch)
    cols = rows // cch
    m = jnp.zeros((h * cch, 8), jnp.float32).at[rows, cols].set(a.reshape(-1))
    return jnp.concatenate([m, m], axis=1)


def kernel(x, edge_index, edge_attr, W1, a_src1, a_dst1, b1,
           W2, a_src2, a_dst2, b2):
    acs = _head_matrix(a_src1)
    acd = _head_matrix(a_dst1)
    a2s = jnp.tile(a_src2.reshape(NCLASS, 1), (1, 16))
    a2d = jnp.tile(a_dst2.reshape(NCLASS, 1), (1, 16))
    s_rows = jnp.arange(16)
    s_cols = jnp.arange(HEADS * HID)
    s_mat = (s_rows[:, None] == (s_cols[None, :] // HID)).astype(jnp.float32)

    loop = jnp.arange(N, dtype=jnp.int32)
    pad = jnp.full((E_PAD - E - N,), N, dtype=jnp.int32)
    src_all = jnp.concatenate([edge_index[0], loop, pad])
    dst_all = jnp.concatenate([edge_index[1], loop, pad])

    h1, as1, ad1 = pl.pallas_call(
        _tc1_body,
        out_shape=[
            jax.ShapeDtypeStruct((N, HEADS * HID), jnp.float32),
            jax.ShapeDtypeStruct((N, 16), jnp.float32),
            jax.ShapeDtypeStruct((N, 16), jnp.float32),
        ],
    )(x, W1, acs, acd)

    hpad = ((0, NPAD - N), (0, 0))
    h1e = jnp.pad(h1, hpad)
    as1e = jnp.pad(as1, hpad, constant_values=_NEG)
    ad1e = jnp.pad(ad1, hpad, constant_values=_NEG)

    num1, den1 = _sc_gat_l1(src_all, dst_all, h1e, as1e, ad1e)

    h2, as2, ad2 = pl.pallas_call(
        _tc2_body,
        out_shape=[
            jax.ShapeDtypeStruct((N, NCLASS), jnp.float32),
            jax.ShapeDtypeStruct((N, 16), jnp.float32),
            jax.ShapeDtypeStruct((N, 16), jnp.float32),
        ],
    )(num1[0, :N], num1[1, :N], den1[0, :N], den1[1, :N],
      s_mat, b1.reshape(1, HEADS * HID), W2, a2s, a2d)

    h2e = jnp.pad(h2, hpad)
    as2e = jnp.pad(as2, hpad, constant_values=_NEG)
    ad2e = jnp.pad(ad2, hpad, constant_values=_NEG)

    num2, den2 = _sc_gat_l2(src_all, dst_all, h2e, as2e, ad2e)

    final, logp = pl.pallas_call(
        _tc3_body,
        out_shape=[
            jax.ShapeDtypeStruct((N, NCLASS), jnp.float32),
            jax.ShapeDtypeStruct((N, NCLASS), jnp.float32),
        ],
    )(num2[0, :N], num2[1, :N], den2[0, :N], den2[1, :N],
      b2.reshape(1, NCLASS))

    return (final, logp)


# speedup vs baseline: 64.7443x; 1.0243x over previous
import functools

import jax
import jax.numpy as jnp
from jax import lax
from jax.experimental import pallas as pl
from jax.experimental.pallas import tpu as pltpu
from jax.experimental.pallas import tpu_sc as plsc

N = 10000
NFEAT = 128
HID = 16
HEADS = 8
NCLASS = 16
E = 320000

NW = 32
CHUNK = 10496
E_PAD = NW * CHUNK
NPAD = 10112
ROWS_PER_TILE = NPAD // 16

_NEG = -1.0e30

_GATHER_DN = lax.GatherDimensionNumbers(
    offset_dims=(), collapsed_slice_dims=(0,), start_index_map=(0,))


def _lane_bcast(v, h):
    idx = jnp.full((16, 1), h, dtype=jnp.int32)
    return lax.gather(v, idx, _GATHER_DN, (1,),
                      mode=lax.GatherScatterMode.PROMISE_IN_BOUNDS)


def _pad_rows(a, value):
    return jnp.pad(a, ((0, NPAD - N), (0, 0)), constant_values=value)


def _tc1_body(x_ref, w1_ref, acs_ref, acd_ref, h_ref, as_ref, ad_ref):
    h = jnp.dot(x_ref[...], w1_ref[...], preferred_element_type=jnp.float32)
    h_ref[...] = _pad_rows(h, 0.0)
    as_ref[...] = _pad_rows(
        jnp.dot(h, acs_ref[...], preferred_element_type=jnp.float32), _NEG)
    ad_ref[...] = _pad_rows(
        jnp.dot(h, acd_ref[...], preferred_element_type=jnp.float32), _NEG)


def _tc2_body(na_ref, nb_ref, da_ref, db_ref, s_ref, b1_ref, w2_ref,
              a2s_ref, a2d_ref, h2_ref, as2_ref, ad2_ref):
    den = da_ref[...] + db_ref[...]
    div = jnp.dot(den, s_ref[...], preferred_element_type=jnp.float32)
    out1 = (na_ref[...] + nb_ref[...]) / div + b1_ref[...]
    x2 = jnp.where(out1 > 0, out1, jnp.exp(out1) - 1.0)
    h2 = jnp.dot(x2, w2_ref[...], preferred_element_type=jnp.float32)
    h2_ref[...] = _pad_rows(h2, 0.0)
    as2_ref[...] = _pad_rows(
        jnp.dot(h2, a2s_ref[...], preferred_element_type=jnp.float32), _NEG)
    ad2_ref[...] = _pad_rows(
        jnp.dot(h2, a2d_ref[...], preferred_element_type=jnp.float32), _NEG)


def _tc3_body(na_ref, nb_ref, da_ref, db_ref, b2_ref, fin_ref, lp_ref):
    fin = (na_ref[...] + nb_ref[...]) / (da_ref[...] + db_ref[...]) + b2_ref[...]
    fin_ref[...] = fin
    m = jnp.max(fin, axis=1, keepdims=True)
    lse = jnp.log(jnp.sum(jnp.exp(fin - m), axis=1, keepdims=True)) + m
    lp_ref[...] = fin - lse


def _make_sc_gat(D, heads, B):
    mesh = plsc.VectorSubcoreMesh(core_axis_name="c", subcore_axis_name="s")
    chan = D // heads
    nb = CHUNK // B
    assert nb * B == CHUNK and nb % 2 == 0
    pairs = nb // 2

    @functools.partial(
        pl.kernel,
        out_type=[
            jax.ShapeDtypeStruct((2, NPAD, D), jnp.float32),
            jax.ShapeDtypeStruct((2, NPAD, 16), jnp.float32),
        ],
        mesh=mesh,
        compiler_params=pltpu.CompilerParams(use_tc_tiling_on_sc=False),
        scratch_types=[
            pltpu.VMEM((B,), jnp.int32),
            pltpu.VMEM((B,), jnp.int32),
            pltpu.VMEM((B, D), jnp.float32),
            pltpu.VMEM((B, D), jnp.float32),
            pltpu.VMEM((B, 16), jnp.float32),
            pltpu.VMEM((B, 16), jnp.float32),
            pltpu.VMEM((B, 16), jnp.float32),
            pltpu.VMEM((B,), jnp.int32),
            pltpu.VMEM((B,), jnp.int32),
            pltpu.VMEM((B, D), jnp.float32),
            pltpu.VMEM((B, D), jnp.float32),
            pltpu.VMEM((B, 16), jnp.float32),
            pltpu.VMEM((B, 16), jnp.float32),
            pltpu.VMEM((B, 16), jnp.float32),
            pltpu.VMEM_SHARED((NPAD, D), jnp.float32),
            pltpu.VMEM_SHARED((NPAD, 16), jnp.float32),
            pltpu.SemaphoreType.DMA,
            pltpu.SemaphoreType.DMA,
            pltpu.SemaphoreType.DMA,
            pltpu.SemaphoreType.DMA,
        ],
    )
    def sc_gat(src_hbm, dst_hbm, h_hbm, as_hbm, ad_hbm, num_hbm, den_hbm,
               src0, dst0, hs0, msg0, as0, ad0, w0,
               src1, dst1, hs1, msg1, as1, ad1, w1,
               num_s, den_s, sg0, sg1, ss0, ss1):
        c = lax.axis_index("c")
        s = lax.axis_index("s")
        wid = c * 16 + s
        slots = ((src0, dst0, hs0, msg0, as0, ad0, w0, sg0, ss0),
                 (src1, dst1, hs1, msg1, as1, ad1, w1, sg1, ss1))

        def prime(S, b):
            sv, dv, hs, msg, asv, adv, wv, sg, ss = S
            base = wid * CHUNK + b * B
            pltpu.sync_copy(src_hbm.at[pl.ds(base, B)], sv)
            pltpu.sync_copy(dst_hbm.at[pl.ds(base, B)], dv)
            pltpu.async_copy(h_hbm.at[sv], hs, sg)
            pltpu.async_copy(as_hbm.at[sv], asv, sg)
            pltpu.async_copy(ad_hbm.at[dv], adv, sg)

        def wait_gathers(S):
            sv, dv, hs, msg, asv, adv, wv, sg, ss = S
            pltpu.make_async_copy(h_hbm.at[sv], hs, sg).wait()
            pltpu.make_async_copy(as_hbm.at[sv], asv, sg).wait()
            pltpu.make_async_copy(ad_hbm.at[dv], adv, sg).wait()

        def scatter(S):
            sv, dv, hs, msg, asv, adv, wv, sg, ss = S
            pltpu.async_copy(msg, num_s.at[dv], ss, add=True)
            pltpu.async_copy(wv, den_s.at[dv], ss, add=True)

        def wait_scatter(S):
            sv, dv, hs, msg, asv, adv, wv, sg, ss = S
            pltpu.make_async_copy(msg, num_s.at[dv], ss).wait()
            pltpu.make_async_copy(wv, den_s.at[dv], ss).wait()

        def compute(S):
            sv, dv, hs, msg, asv, adv, wv, sg, ss = S

            @plsc.parallel_loop(0, B, 1, unroll=8)
            def _edge(i):
                e = asv[i] + adv[i]
                e = jnp.maximum(e, 0.2 * e)
                w = jnp.exp(e)
                wv[i] = w
                if heads == 1:
                    msg[i] = hs[i] * w
                else:
                    for h in range(heads):
                        wh = _lane_bcast(w, h)
                        msg[i, pl.ds(h * chan, chan)] = (
                            hs[i, pl.ds(h * chan, chan)] * wh)

        def _zrow(i, _):
            zero = jnp.zeros((16,), jnp.float32)
            for j in range(D // 16):
                msg0[i, pl.ds(j * 16, 16)] = zero
            w0[i, pl.ds(0, 16)] = zero
            return 0
        lax.fori_loop(0, B, _zrow, 0)
        rb = s * ROWS_PER_TILE
        full, rem = ROWS_PER_TILE // B, ROWS_PER_TILE % B
        for k in range(full):
            pltpu.sync_copy(msg0, num_s.at[pl.ds(rb + k * B, B)])
            pltpu.sync_copy(w0, den_s.at[pl.ds(rb + k * B, B)])
        if rem:
            pltpu.sync_copy(msg0.at[pl.ds(0, rem)],
                            num_s.at[pl.ds(rb + full * B, rem)])
            pltpu.sync_copy(w0.at[pl.ds(0, rem)],
                            den_s.at[pl.ds(rb + full * B, rem)])
        prime(slots[0], 0)
        prime(slots[1], 1)
        plsc.subcore_barrier()

        def _pair(p, _):
            b0 = 2 * p
            wait_gathers(slots[0])
            compute(slots[0])
            scatter(slots[0])
            wait_gathers(slots[1])
            compute(slots[1])
            scatter(slots[1])

            @pl.when(p < pairs - 1)
            def _():
                wait_scatter(slots[0])
                prime(slots[0], b0 + 2)
                wait_scatter(slots[1])
                prime(slots[1], b0 + 3)
            return 0
        lax.fori_loop(0, pairs, _pair, 0)
        wait_scatter(slots[0])
        wait_scatter(slots[1])

        plsc.subcore_barrier()
        pltpu.sync_copy(num_s.at[pl.ds(rb, ROWS_PER_TILE)],
                        num_hbm.at[c, pl.ds(rb, ROWS_PER_TILE)])
        pltpu.sync_copy(den_s.at[pl.ds(rb, ROWS_PER_TILE)],
                        den_hbm.at[c, pl.ds(rb, ROWS_PER_TILE)])

    return sc_gat


_sc_gat_l1 = _make_sc_gat(HEADS * HID, HEADS, 64)
_sc_gat_l2 = _make_sc_gat(NCLASS, 1, 128)


def _head_matrix(a):
    h, cch = a.shape
    rows = jnp.arange(h * cch)
    cols = rows // cch
    m = jnp.zeros((h * cch, 8), jnp.float32).at[rows, cols].set(a.reshape(-1))
    return jnp.concatenate([m, m], axis=1)


def kernel(x, edge_index, edge_attr, W1, a_src1, a_dst1, b1,
           W2, a_src2, a_dst2, b2):
    acs = _head_matrix(a_src1)
    acd = _head_matrix(a_dst1)
    a2s = jnp.tile(a_src2.reshape(NCLASS, 1), (1, 16))
    a2d = jnp.tile(a_dst2.reshape(NCLASS, 1), (1, 16))
    s_rows = jnp.arange(16)
    s_cols = jnp.arange(HEADS * HID)
    s_mat = (s_rows[:, None] == (s_cols[None, :] // HID)).astype(jnp.float32)

    loop = jnp.arange(N, dtype=jnp.int32)
    pad = jnp.full((E_PAD - E - N,), N, dtype=jnp.int32)
    src_all = jnp.concatenate([edge_index[0], loop, pad])
    dst_all = jnp.concatenate([edge_index[1], loop, pad])

    h1, as1, ad1 = pl.pallas_call(
        _tc1_body,
        out_shape=[
            jax.ShapeDtypeStruct((NPAD, HEADS * HID), jnp.float32),
            jax.ShapeDtypeStruct((NPAD, 16), jnp.float32),
            jax.ShapeDtypeStruct((NPAD, 16), jnp.float32),
        ],
    )(x, W1, acs, acd)

    num1, den1 = _sc_gat_l1(src_all, dst_all, h1, as1, ad1)

    h2, as2, ad2 = pl.pallas_call(
        _tc2_body,
        out_shape=[
            jax.ShapeDtypeStruct((NPAD, NCLASS), jnp.float32),
            jax.ShapeDtypeStruct((NPAD, 16), jnp.float32),
            jax.ShapeDtypeStruct((NPAD, 16), jnp.float32),
        ],
    )(num1[0, :N], num1[1, :N], den1[0, :N], den1[1, :N],
      s_mat, b1.reshape(1, HEADS * HID), W2, a2s, a2d)

    num2, den2 = _sc_gat_l2(src_all, dst_all, h2, as2, ad2)

    final, logp = pl.pallas_call(
        _tc3_body,
        out_shape=[
            jax.ShapeDtypeStruct((N, NCLASS), jnp.float32),
            jax.ShapeDtypeStruct((N, NCLASS), jnp.float32),
        ],
    )(num2[0, :N], num2[1, :N], den2[0, :N], den2[1, :N],
      b2.reshape(1, NCLASS))

    return (final, logp)


# speedup vs baseline: 64.7879x; 1.0007x over previous
import functools

import jax
import jax.numpy as jnp
from jax import lax
from jax.experimental import pallas as pl
from jax.experimental.pallas import tpu as pltpu
from jax.experimental.pallas import tpu_sc as plsc

N = 10000
NFEAT = 128
HID = 16
HEADS = 8
NCLASS = 16
E = 320000

NW = 32
CHUNK = 10496
E_PAD = NW * CHUNK
NPAD = 10112
ROWS_PER_TILE = NPAD // 16

_NEG = -1.0e30

_GATHER_DN = lax.GatherDimensionNumbers(
    offset_dims=(), collapsed_slice_dims=(0,), start_index_map=(0,))


def _lane_bcast(v, h):
    idx = jnp.full((16, 1), h, dtype=jnp.int32)
    return lax.gather(v, idx, _GATHER_DN, (1,),
                      mode=lax.GatherScatterMode.PROMISE_IN_BOUNDS)


def _pad_rows(a, value):
    return jnp.pad(a, ((0, NPAD - N), (0, 0)), constant_values=value)


def _tc1_body(x_ref, w1_ref, acs_ref, acd_ref, h_ref, as_ref, ad_ref):
    h = jnp.dot(x_ref[...], w1_ref[...], preferred_element_type=jnp.float32)
    h_ref[...] = _pad_rows(h, 0.0)
    as_ref[...] = _pad_rows(
        jnp.dot(h, acs_ref[...], preferred_element_type=jnp.float32), _NEG)
    ad_ref[...] = _pad_rows(
        jnp.dot(h, acd_ref[...], preferred_element_type=jnp.float32), _NEG)


def _tc2_body(na_ref, nb_ref, da_ref, db_ref, s_ref, b1_ref, w2_ref,
              a2s_ref, a2d_ref, h2_ref, as2_ref, ad2_ref):
    den = da_ref[...] + db_ref[...]
    div = jnp.dot(den, s_ref[...], preferred_element_type=jnp.float32)
    out1 = (na_ref[...] + nb_ref[...]) / div + b1_ref[...]
    x2 = jnp.where(out1 > 0, out1, jnp.exp(out1) - 1.0)
    h2 = jnp.dot(x2, w2_ref[...], preferred_element_type=jnp.float32)
    h2_ref[...] = _pad_rows(h2, 0.0)
    as2_ref[...] = _pad_rows(
        jnp.dot(h2, a2s_ref[...], preferred_element_type=jnp.float32), _NEG)
    ad2_ref[...] = _pad_rows(
        jnp.dot(h2, a2d_ref[...], preferred_element_type=jnp.float32), _NEG)


def _tc3_body(na_ref, nb_ref, da_ref, db_ref, b2_ref, fin_ref, lp_ref):
    fin = (na_ref[...] + nb_ref[...]) / (da_ref[...] + db_ref[...]) + b2_ref[...]
    fin_ref[...] = fin
    m = jnp.max(fin, axis=1, keepdims=True)
    lse = jnp.log(jnp.sum(jnp.exp(fin - m), axis=1, keepdims=True)) + m
    lp_ref[...] = fin - lse


def _make_sc_gat(D, heads, B):
    mesh = plsc.VectorSubcoreMesh(core_axis_name="c", subcore_axis_name="s")
    chan = D // heads
    nb = CHUNK // B
    assert nb * B == CHUNK and nb % 2 == 0
    pairs = nb // 2

    @functools.partial(
        pl.kernel,
        out_type=[
            jax.ShapeDtypeStruct((2, NPAD, D), jnp.float32),
            jax.ShapeDtypeStruct((2, NPAD, 16), jnp.float32),
        ],
        mesh=mesh,
        compiler_params=pltpu.CompilerParams(use_tc_tiling_on_sc=False),
        scratch_types=[
            pltpu.VMEM((B,), jnp.int32),
            pltpu.VMEM((B,), jnp.int32),
            pltpu.VMEM((B, D), jnp.float32),
            pltpu.VMEM((B, D), jnp.float32),
            pltpu.VMEM((B, 16), jnp.float32),
            pltpu.VMEM((B, 16), jnp.float32),
            pltpu.VMEM((B, 16), jnp.float32),
            pltpu.VMEM((B,), jnp.int32),
            pltpu.VMEM((B,), jnp.int32),
            pltpu.VMEM((B, D), jnp.float32),
            pltpu.VMEM((B, D), jnp.float32),
            pltpu.VMEM((B, 16), jnp.float32),
            pltpu.VMEM((B, 16), jnp.float32),
            pltpu.VMEM((B, 16), jnp.float32),
            pltpu.VMEM_SHARED((NPAD, D), jnp.float32),
            pltpu.VMEM_SHARED((NPAD, 16), jnp.float32),
            pltpu.SemaphoreType.DMA,
            pltpu.SemaphoreType.DMA,
            pltpu.SemaphoreType.DMA,
            pltpu.SemaphoreType.DMA,
        ],
    )
    def sc_gat(src_hbm, dst_hbm, h_hbm, as_hbm, ad_hbm, num_hbm, den_hbm,
               src0, dst0, hs0, msg0, as0, ad0, w0,
               src1, dst1, hs1, msg1, as1, ad1, w1,
               num_s, den_s, sg0, sg1, ss0, ss1):
        c = lax.axis_index("c")
        s = lax.axis_index("s")
        wid = s * 2 + c
        slots = ((src0, dst0, hs0, msg0, as0, ad0, w0, sg0, ss0),
                 (src1, dst1, hs1, msg1, as1, ad1, w1, sg1, ss1))

        def prime(S, b):
            sv, dv, hs, msg, asv, adv, wv, sg, ss = S
            base = wid * CHUNK + b * B
            pltpu.sync_copy(src_hbm.at[pl.ds(base, B)], sv)
            pltpu.sync_copy(dst_hbm.at[pl.ds(base, B)], dv)
            pltpu.async_copy(h_hbm.at[sv], hs, sg)
            pltpu.async_copy(as_hbm.at[sv], asv, sg)
            pltpu.async_copy(ad_hbm.at[dv], adv, sg)

        def wait_gathers(S):
            sv, dv, hs, msg, asv, adv, wv, sg, ss = S
            pltpu.make_async_copy(h_hbm.at[sv], hs, sg).wait()
            pltpu.make_async_copy(as_hbm.at[sv], asv, sg).wait()
            pltpu.make_async_copy(ad_hbm.at[dv], adv, sg).wait()

        def scatter(S):
            sv, dv, hs, msg, asv, adv, wv, sg, ss = S
            pltpu.async_copy(msg, num_s.at[dv], ss, add=True)
            pltpu.async_copy(wv, den_s.at[dv], ss, add=True)

        def wait_scatter(S):
            sv, dv, hs, msg, asv, adv, wv, sg, ss = S
            pltpu.make_async_copy(msg, num_s.at[dv], ss).wait()
            pltpu.make_async_copy(wv, den_s.at[dv], ss).wait()

        def compute(S):
            sv, dv, hs, msg, asv, adv, wv, sg, ss = S

            @plsc.parallel_loop(0, B, 1, unroll=8)
            def _edge(i):
                e = asv[i] + adv[i]
                e = jnp.maximum(e, 0.2 * e)
                w = jnp.exp(e)
                wv[i] = w
                if heads == 1:
                    msg[i] = hs[i] * w
                else:
                    for h in range(heads):
                        wh = _lane_bcast(w, h)
                        msg[i, pl.ds(h * chan, chan)] = (
                            hs[i, pl.ds(h * chan, chan)] * wh)

        def _zrow(i, _):
            zero = jnp.zeros((16,), jnp.float32)
            for j in range(D // 16):
                msg0[i, pl.ds(j * 16, 16)] = zero
            w0[i, pl.ds(0, 16)] = zero
            return 0
        lax.fori_loop(0, B, _zrow, 0)
        rb = s * ROWS_PER_TILE
        full, rem = ROWS_PER_TILE // B, ROWS_PER_TILE % B
        for k in range(full):
            pltpu.sync_copy(msg0, num_s.at[pl.ds(rb + k * B, B)])
            pltpu.sync_copy(w0, den_s.at[pl.ds(rb + k * B, B)])
        if rem:
            pltpu.sync_copy(msg0.at[pl.ds(0, rem)],
                            num_s.at[pl.ds(rb + full * B, rem)])
            pltpu.sync_copy(w0.at[pl.ds(0, rem)],
                            den_s.at[pl.ds(rb + full * B, rem)])
        prime(slots[0], 0)
        prime(slots[1], 1)
        plsc.subcore_barrier()

        def _pair(p, _):
            b0 = 2 * p
            wait_gathers(slots[0])
            compute(slots[0])
            scatter(slots[0])
            wait_gathers(slots[1])
            compute(slots[1])
            scatter(slots[1])

            @pl.when(p < pairs - 1)
            def _():
                wait_scatter(slots[0])
                prime(slots[0], b0 + 2)
                wait_scatter(slots[1])
                prime(slots[1], b0 + 3)
            return 0
        lax.fori_loop(0, pairs, _pair, 0)
        wait_scatter(slots[0])
        wait_scatter(slots[1])

        plsc.subcore_barrier()
        pltpu.sync_copy(num_s.at[pl.ds(rb, ROWS_PER_TILE)],
                        num_hbm.at[c, pl.ds(rb, ROWS_PER_TILE)])
        pltpu.sync_copy(den_s.at[pl.ds(rb, ROWS_PER_TILE)],
                        den_hbm.at[c, pl.ds(rb, ROWS_PER_TILE)])

    return sc_gat


_sc_gat_l1 = _make_sc_gat(HEADS * HID, HEADS, 64)
_sc_gat_l2 = _make_sc_gat(NCLASS, 1, 128)


def _head_matrix(a):
    h, cch = a.shape
    rows = jnp.arange(h * cch)
    cols = rows // cch
    m = jnp.zeros((h * cch, 8), jnp.float32).at[rows, cols].set(a.reshape(-1))
    return jnp.concatenate([m, m], axis=1)


def kernel(x, edge_index, edge_attr, W1, a_src1, a_dst1, b1,
           W2, a_src2, a_dst2, b2):
    acs = _head_matrix(a_src1)
    acd = _head_matrix(a_dst1)
    a2s = jnp.tile(a_src2.reshape(NCLASS, 1), (1, 16))
    a2d = jnp.tile(a_dst2.reshape(NCLASS, 1), (1, 16))
    s_rows = jnp.arange(16)
    s_cols = jnp.arange(HEADS * HID)
    s_mat = (s_rows[:, None] == (s_cols[None, :] // HID)).astype(jnp.float32)

    loop = jnp.arange(N, dtype=jnp.int32)
    pad = jnp.full((E_PAD - E - N,), N, dtype=jnp.int32)
    src_all = jnp.concatenate([edge_index[0], loop, pad])
    dst_all = jnp.concatenate([edge_index[1], loop, pad])

    h1, as1, ad1 = pl.pallas_call(
        _tc1_body,
        out_shape=[
            jax.ShapeDtypeStruct((NPAD, HEADS * HID), jnp.float32),
            jax.ShapeDtypeStruct((NPAD, 16), jnp.float32),
            jax.ShapeDtypeStruct((NPAD, 16), jnp.float32),
        ],
    )(x, W1, acs, acd)

    num1, den1 = _sc_gat_l1(src_all, dst_all, h1, as1, ad1)

    h2, as2, ad2 = pl.pallas_call(
        _tc2_body,
        out_shape=[
            jax.ShapeDtypeStruct((NPAD, NCLASS), jnp.float32),
            jax.ShapeDtypeStruct((NPAD, 16), jnp.float32),
            jax.ShapeDtypeStruct((NPAD, 16), jnp.float32),
        ],
    )(num1[0, :N], num1[1, :N], den1[0, :N], den1[1, :N],
      s_mat, b1.reshape(1, HEADS * HID), W2, a2s, a2d)

    num2, den2 = _sc_gat_l2(src_all, dst_all, h2, as2, ad2)

    final, logp = pl.pallas_call(
        _tc3_body,
        out_shape=[
            jax.ShapeDtypeStruct((N, NCLASS), jnp.float32),
            jax.ShapeDtypeStruct((N, NCLASS), jnp.float32),
        ],
    )(num2[0, :N], num2[1, :N], den2[0, :N], den2[1, :N],
      b2.reshape(1, NCLASS))

    return (final, logp)


# speedup vs baseline: 76.3000x; 1.1777x over previous
import functools

import jax
import jax.numpy as jnp
from jax import lax
from jax.experimental import pallas as pl
from jax.experimental.pallas import tpu as pltpu
from jax.experimental.pallas import tpu_sc as plsc

N = 10000
NFEAT = 128
HID = 16
HEADS = 8
NCLASS = 16
E = 320000

NW = 32
CHUNK = 10496
E_PAD = NW * CHUNK
NPAD = 10112
ROWS_PER_TILE = NPAD // 16

_NEG = -1.0e30

_GATHER_DN = lax.GatherDimensionNumbers(
    offset_dims=(), collapsed_slice_dims=(0,), start_index_map=(0,))


def _lane_bcast(v, h):
    idx = jnp.full((16, 1), h, dtype=jnp.int32)
    return lax.gather(v, idx, _GATHER_DN, (1,),
                      mode=lax.GatherScatterMode.PROMISE_IN_BOUNDS)


def _pad_rows(a, value):
    return jnp.pad(a, ((0, NPAD - N), (0, 0)), constant_values=value)


def _tc1_body(x_ref, w1_ref, acs_ref, acd_ref, h_ref, as_ref, ad_ref):
    h = jnp.dot(x_ref[...], w1_ref[...], preferred_element_type=jnp.float32)
    h_ref[...] = _pad_rows(h, 0.0)
    as_ref[...] = _pad_rows(
        jnp.dot(h, acs_ref[...], preferred_element_type=jnp.float32), _NEG)
    ad_ref[...] = _pad_rows(
        jnp.dot(h, acd_ref[...], preferred_element_type=jnp.float32), _NEG)


def _tc2_body(num_ref, den_ref, s_ref, b1_ref, w2_ref,
              a2s_ref, a2d_ref, h2_ref, as2_ref, ad2_ref):
    den = den_ref[0, :N] + den_ref[1, :N]
    div = jnp.dot(den, s_ref[...], preferred_element_type=jnp.float32)
    out1 = (num_ref[0, :N] + num_ref[1, :N]) / div + b1_ref[...]
    x2 = jnp.where(out1 > 0, out1, jnp.exp(out1) - 1.0)
    h2 = jnp.dot(x2, w2_ref[...], preferred_element_type=jnp.float32)
    h2_ref[...] = _pad_rows(h2, 0.0)
    as2_ref[...] = _pad_rows(
        jnp.dot(h2, a2s_ref[...], preferred_element_type=jnp.float32), _NEG)
    ad2_ref[...] = _pad_rows(
        jnp.dot(h2, a2d_ref[...], preferred_element_type=jnp.float32), _NEG)


def _tc3_body(num_ref, den_ref, b2_ref, fin_ref, lp_ref):
    fin = ((num_ref[0, :N] + num_ref[1, :N])
           / (den_ref[0, :N] + den_ref[1, :N]) + b2_ref[...])
    fin_ref[...] = fin
    m = jnp.max(fin, axis=1, keepdims=True)
    lse = jnp.log(jnp.sum(jnp.exp(fin - m), axis=1, keepdims=True)) + m
    lp_ref[...] = fin - lse


def _make_sc_gat(D, heads, B, chunk0, chunk1):
    mesh = plsc.VectorSubcoreMesh(core_axis_name="c", subcore_axis_name="s")
    chan = D // heads
    assert chunk0 + chunk1 == 2 * CHUNK
    nb0, nb1 = chunk0 // B, chunk1 // B
    assert nb0 * B == chunk0 and nb1 * B == chunk1
    assert nb0 % 2 == 0 and nb1 % 2 == 0
    pairs0, pairs1 = nb0 // 2, nb1 // 2
    off1 = 16 * chunk0

    @functools.partial(
        pl.kernel,
        out_type=[
            jax.ShapeDtypeStruct((2, NPAD, D), jnp.float32),
            jax.ShapeDtypeStruct((2, NPAD, 16), jnp.float32),
        ],
        mesh=mesh,
        compiler_params=pltpu.CompilerParams(use_tc_tiling_on_sc=False),
        scratch_types=[
            pltpu.VMEM((B,), jnp.int32),
            pltpu.VMEM((B,), jnp.int32),
            pltpu.VMEM((B, D), jnp.float32),
            pltpu.VMEM((B, D), jnp.float32),
            pltpu.VMEM((B, 16), jnp.float32),
            pltpu.VMEM((B, 16), jnp.float32),
            pltpu.VMEM((B, 16), jnp.float32),
            pltpu.VMEM((B,), jnp.int32),
            pltpu.VMEM((B,), jnp.int32),
            pltpu.VMEM((B, D), jnp.float32),
            pltpu.VMEM((B, D), jnp.float32),
            pltpu.VMEM((B, 16), jnp.float32),
            pltpu.VMEM((B, 16), jnp.float32),
            pltpu.VMEM((B, 16), jnp.float32),
            pltpu.VMEM_SHARED((NPAD, D), jnp.float32),
            pltpu.VMEM_SHARED((NPAD, 16), jnp.float32),
            pltpu.SemaphoreType.DMA,
            pltpu.SemaphoreType.DMA,
            pltpu.SemaphoreType.DMA,
            pltpu.SemaphoreType.DMA,
        ],
    )
    def sc_gat(src_hbm, dst_hbm, h_hbm, as_hbm, ad_hbm, num_hbm, den_hbm,
               src0, dst0, hs0, msg0, as0, ad0, w0,
               src1, dst1, hs1, msg1, as1, ad1, w1,
               num_s, den_s, sg0, sg1, ss0, ss1):
        c = lax.axis_index("c")
        s = lax.axis_index("s")
        tile_base = jnp.where(c == 0, s * chunk0, off1 + s * chunk1)
        my_pairs = jnp.where(c == 0, pairs0, pairs1)
        slots = ((src0, dst0, hs0, msg0, as0, ad0, w0, sg0, ss0),
                 (src1, dst1, hs1, msg1, as1, ad1, w1, sg1, ss1))

        def prime(S, b):
            sv, dv, hs, msg, asv, adv, wv, sg, ss = S
            base = tile_base + b * B
            pltpu.sync_copy(src_hbm.at[pl.ds(base, B)], sv)
            pltpu.sync_copy(dst_hbm.at[pl.ds(base, B)], dv)
            pltpu.async_copy(h_hbm.at[sv], hs, sg)
            pltpu.async_copy(as_hbm.at[sv], asv, sg)
            pltpu.async_copy(ad_hbm.at[dv], adv, sg)

        def wait_gathers(S):
            sv, dv, hs, msg, asv, adv, wv, sg, ss = S
            pltpu.make_async_copy(h_hbm.at[sv], hs, sg).wait()
            pltpu.make_async_copy(as_hbm.at[sv], asv, sg).wait()
            pltpu.make_async_copy(ad_hbm.at[dv], adv, sg).wait()

        def scatter(S):
            sv, dv, hs, msg, asv, adv, wv, sg, ss = S
            pltpu.async_copy(msg, num_s.at[dv], ss, add=True)
            pltpu.async_copy(wv, den_s.at[dv], ss, add=True)

        def wait_scatter(S):
            sv, dv, hs, msg, asv, adv, wv, sg, ss = S
            pltpu.make_async_copy(msg, num_s.at[dv], ss).wait()
            pltpu.make_async_copy(wv, den_s.at[dv], ss).wait()

        def compute(S):
            sv, dv, hs, msg, asv, adv, wv, sg, ss = S

            @plsc.parallel_loop(0, B, 1, unroll=8)
            def _edge(i):
                e = asv[i] + adv[i]
                e = jnp.maximum(e, 0.2 * e)
                w = jnp.exp(e)
                wv[i] = w
                if heads == 1:
                    msg[i] = hs[i] * w
                else:
                    for h in range(heads):
                        wh = _lane_bcast(w, h)
                        msg[i, pl.ds(h * chan, chan)] = (
                            hs[i, pl.ds(h * chan, chan)] * wh)

        def _zrow(i, _):
            zero = jnp.zeros((16,), jnp.float32)
            for j in range(D // 16):
                msg0[i, pl.ds(j * 16, 16)] = zero
            w0[i, pl.ds(0, 16)] = zero
            return 0
        lax.fori_loop(0, B, _zrow, 0)
        rb = s * ROWS_PER_TILE
        full, rem = ROWS_PER_TILE // B, ROWS_PER_TILE % B
        for k in range(full):
            pltpu.sync_copy(msg0, num_s.at[pl.ds(rb + k * B, B)])
            pltpu.sync_copy(w0, den_s.at[pl.ds(rb + k * B, B)])
        if rem:
            pltpu.sync_copy(msg0.at[pl.ds(0, rem)],
                            num_s.at[pl.ds(rb + full * B, rem)])
            pltpu.sync_copy(w0.at[pl.ds(0, rem)],
                            den_s.at[pl.ds(rb + full * B, rem)])
        prime(slots[0], 0)
        prime(slots[1], 1)
        plsc.subcore_barrier()

        def _pair(p, _):
            b0 = 2 * p
            wait_gathers(slots[0])
            compute(slots[0])
            scatter(slots[0])
            wait_gathers(slots[1])
            compute(slots[1])
            scatter(slots[1])

            @pl.when(p < my_pairs - 1)
            def _():
                wait_scatter(slots[0])
                prime(slots[0], b0 + 2)
                wait_scatter(slots[1])
                prime(slots[1], b0 + 3)
            return 0
        lax.fori_loop(0, my_pairs, _pair, 0)
        wait_scatter(slots[0])
        wait_scatter(slots[1])

        plsc.subcore_barrier()
        pltpu.sync_copy(num_s.at[pl.ds(rb, ROWS_PER_TILE)],
                        num_hbm.at[c, pl.ds(rb, ROWS_PER_TILE)])
        pltpu.sync_copy(den_s.at[pl.ds(rb, ROWS_PER_TILE)],
                        den_hbm.at[c, pl.ds(rb, ROWS_PER_TILE)])

    return sc_gat


_sc_gat_l1 = _make_sc_gat(HEADS * HID, HEADS, 64, 13184, 7808)
_sc_gat_l2 = _make_sc_gat(NCLASS, 1, 128, CHUNK, CHUNK)


def _head_matrix(a):
    h, cch = a.shape
    rows = jnp.arange(h * cch)
    cols = rows // cch
    m = jnp.zeros((h * cch, 8), jnp.float32).at[rows, cols].set(a.reshape(-1))
    return jnp.concatenate([m, m], axis=1)


def kernel(x, edge_index, edge_attr, W1, a_src1, a_dst1, b1,
           W2, a_src2, a_dst2, b2):
    acs = _head_matrix(a_src1)
    acd = _head_matrix(a_dst1)
    a2s = jnp.tile(a_src2.reshape(NCLASS, 1), (1, 16))
    a2d = jnp.tile(a_dst2.reshape(NCLASS, 1), (1, 16))
    s_rows = jnp.arange(16)
    s_cols = jnp.arange(HEADS * HID)
    s_mat = (s_rows[:, None] == (s_cols[None, :] // HID)).astype(jnp.float32)

    loop = jnp.arange(N, dtype=jnp.int32)
    pad = jnp.full((E_PAD - E - N,), N, dtype=jnp.int32)
    src_all = jnp.concatenate([edge_index[0], loop, pad])
    dst_all = jnp.concatenate([edge_index[1], loop, pad])

    h1, as1, ad1 = pl.pallas_call(
        _tc1_body,
        out_shape=[
            jax.ShapeDtypeStruct((NPAD, HEADS * HID), jnp.float32),
            jax.ShapeDtypeStruct((NPAD, 16), jnp.float32),
            jax.ShapeDtypeStruct((NPAD, 16), jnp.float32),
        ],
    )(x, W1, acs, acd)

    num1, den1 = _sc_gat_l1(src_all, dst_all, h1, as1, ad1)

    h2, as2, ad2 = pl.pallas_call(
        _tc2_body,
        out_shape=[
            jax.ShapeDtypeStruct((NPAD, NCLASS), jnp.float32),
            jax.ShapeDtypeStruct((NPAD, 16), jnp.float32),
            jax.ShapeDtypeStruct((NPAD, 16), jnp.float32),
        ],
    )(num1, den1, s_mat, b1.reshape(1, HEADS * HID), W2, a2s, a2d)

    num2, den2 = _sc_gat_l2(src_all, dst_all, h2, as2, ad2)

    final, logp = pl.pallas_call(
        _tc3_body,
        out_shape=[
            jax.ShapeDtypeStruct((N, NCLASS), jnp.float32),
            jax.ShapeDtypeStruct((N, NCLASS), jnp.float32),
        ],
    )(num2, den2, b2.reshape(1, NCLASS))

    return (final, logp)


# speedup vs baseline: 83.1158x; 1.0893x over previous
import functools

import jax
import jax.numpy as jnp
from jax import lax
from jax.experimental import pallas as pl
from jax.experimental.pallas import tpu as pltpu
from jax.experimental.pallas import tpu_sc as plsc

N = 10000
NFEAT = 128
HID = 16
HEADS = 8
NCLASS = 16
E = 320000

NW = 32
CHUNK = 10496
E_PAD = NW * CHUNK
NPAD = 10112
ROWS_PER_TILE = NPAD // 16

_NEG = -1.0e30

_GATHER_DN = lax.GatherDimensionNumbers(
    offset_dims=(), collapsed_slice_dims=(0,), start_index_map=(0,))


def _lane_bcast(v, h):
    idx = jnp.full((16, 1), h, dtype=jnp.int32)
    return lax.gather(v, idx, _GATHER_DN, (1,),
                      mode=lax.GatherScatterMode.PROMISE_IN_BOUNDS)


def _pad_rows(a, value):
    return jnp.pad(a, ((0, NPAD - N), (0, 0)), constant_values=value)


def _tc1_body(x_ref, w1_ref, acs_ref, acd_ref, h_ref, as_ref, ad_ref):
    h = jnp.dot(x_ref[...], w1_ref[...], preferred_element_type=jnp.float32)
    h_ref[...] = _pad_rows(h, 0.0)
    as_ref[...] = _pad_rows(
        jnp.dot(h, acs_ref[...], preferred_element_type=jnp.float32), _NEG)
    ad_ref[...] = _pad_rows(
        jnp.dot(h, acd_ref[...], preferred_element_type=jnp.float32), _NEG)


def _tc2_body(num_ref, den_ref, s_ref, b1_ref, w2_ref,
              a2s_ref, a2d_ref, h2_ref, as2_ref, ad2_ref):
    den = den_ref[0, :N] + den_ref[1, :N]
    div = jnp.dot(den, s_ref[...], preferred_element_type=jnp.float32)
    out1 = (num_ref[0, :N] + num_ref[1, :N]) / div + b1_ref[...]
    x2 = jnp.where(out1 > 0, out1, jnp.exp(out1) - 1.0)
    h2 = jnp.dot(x2, w2_ref[...], preferred_element_type=jnp.float32)
    h2_ref[...] = _pad_rows(h2, 0.0)
    as2_ref[...] = _pad_rows(
        jnp.dot(h2, a2s_ref[...], preferred_element_type=jnp.float32), _NEG)
    ad2_ref[...] = _pad_rows(
        jnp.dot(h2, a2d_ref[...], preferred_element_type=jnp.float32), _NEG)


def _tc3_body(num_ref, den_ref, b2_ref, fin_ref, lp_ref):
    fin = ((num_ref[0, :N] + num_ref[1, :N])
           / (den_ref[0, :N] + den_ref[1, :N]) + b2_ref[...])
    fin_ref[...] = fin
    m = jnp.max(fin, axis=1, keepdims=True)
    lse = jnp.log(jnp.sum(jnp.exp(fin - m), axis=1, keepdims=True)) + m
    lp_ref[...] = fin - lse


def _make_sc_gat(D, heads, B, chunk0, chunk1, packed=False):
    mesh = plsc.VectorSubcoreMesh(core_axis_name="c", subcore_axis_name="s")
    chan = D // heads
    assert chunk0 + chunk1 == 2 * CHUNK
    nb0, nb1 = chunk0 // B, chunk1 // B
    assert nb0 * B == chunk0 and nb1 * B == chunk1
    assert nb0 % 2 == 0 and nb1 % 2 == 0
    pairs0, pairs1 = nb0 // 2, nb1 // 2
    off1 = 16 * chunk0

    @functools.partial(
        pl.kernel,
        out_type=[
            jax.ShapeDtypeStruct((2, NPAD, D), jnp.float32),
            jax.ShapeDtypeStruct((2, NPAD, 16), jnp.float32),
        ],
        mesh=mesh,
        compiler_params=pltpu.CompilerParams(use_tc_tiling_on_sc=False, needs_layout_passes=False),
        scratch_types=[
            pltpu.VMEM((B,), jnp.int32),
            pltpu.VMEM((B,), jnp.int32),
            (pltpu.VMEM((B, D // 2), jnp.uint32) if packed
             else pltpu.VMEM((B, D), jnp.float32)),
            pltpu.VMEM((B, D), jnp.float32),
            pltpu.VMEM((B, 16), jnp.float32),
            pltpu.VMEM((B, 16), jnp.float32),
            pltpu.VMEM((B, 16), jnp.float32),
            pltpu.VMEM((B,), jnp.int32),
            pltpu.VMEM((B,), jnp.int32),
            (pltpu.VMEM((B, D // 2), jnp.uint32) if packed
             else pltpu.VMEM((B, D), jnp.float32)),
            pltpu.VMEM((B, D), jnp.float32),
            pltpu.VMEM((B, 16), jnp.float32),
            pltpu.VMEM((B, 16), jnp.float32),
            pltpu.VMEM((B, 16), jnp.float32),
            pltpu.VMEM_SHARED((NPAD, D), jnp.float32),
            pltpu.VMEM_SHARED((NPAD, 16), jnp.float32),
            pltpu.SemaphoreType.DMA,
            pltpu.SemaphoreType.DMA,
            pltpu.SemaphoreType.DMA,
            pltpu.SemaphoreType.DMA,
        ],
    )
    def sc_gat(src_hbm, dst_hbm, h_hbm, as_hbm, ad_hbm, num_hbm, den_hbm,
               src0, dst0, hs0, msg0, as0, ad0, w0,
               src1, dst1, hs1, msg1, as1, ad1, w1,
               num_s, den_s, sg0, sg1, ss0, ss1):
        c = lax.axis_index("c")
        s = lax.axis_index("s")
        tile_base = jnp.where(c == 0, s * chunk0, off1 + s * chunk1)
        my_pairs = jnp.where(c == 0, pairs0, pairs1)
        slots = ((src0, dst0, hs0, msg0, as0, ad0, w0, sg0, ss0),
                 (src1, dst1, hs1, msg1, as1, ad1, w1, sg1, ss1))

        def prime(S, b):
            sv, dv, hs, msg, asv, adv, wv, sg, ss = S
            base = tile_base + b * B
            pltpu.sync_copy(src_hbm.at[pl.ds(base, B)], sv)
            pltpu.sync_copy(dst_hbm.at[pl.ds(base, B)], dv)
            pltpu.async_copy(h_hbm.at[sv], hs, sg)
            pltpu.async_copy(as_hbm.at[sv], asv, sg)
            pltpu.async_copy(ad_hbm.at[dv], adv, sg)

        def wait_gathers(S):
            sv, dv, hs, msg, asv, adv, wv, sg, ss = S
            pltpu.make_async_copy(h_hbm.at[sv], hs, sg).wait()
            pltpu.make_async_copy(as_hbm.at[sv], asv, sg).wait()
            pltpu.make_async_copy(ad_hbm.at[dv], adv, sg).wait()

        def scatter(S):
            sv, dv, hs, msg, asv, adv, wv, sg, ss = S
            pltpu.async_copy(msg, num_s.at[dv], ss, add=True)
            pltpu.async_copy(wv, den_s.at[dv], ss, add=True)

        def wait_scatter(S):
            sv, dv, hs, msg, asv, adv, wv, sg, ss = S
            pltpu.make_async_copy(msg, num_s.at[dv], ss).wait()
            pltpu.make_async_copy(wv, den_s.at[dv], ss).wait()

        def compute(S):
            sv, dv, hs, msg, asv, adv, wv, sg, ss = S

            @plsc.parallel_loop(0, B, 1, unroll=8)
            def _edge(i):
                e = asv[i] + adv[i]
                e = jnp.maximum(e, 0.2 * e)
                w = jnp.exp(e)
                wv[i] = w
                if heads == 1:
                    msg[i] = hs[i] * w
                elif packed:
                    for g in range(heads // 2):
                        wrd = hs[i, pl.ds(g * chan, chan)]
                        lo = plsc.bitcast(wrd << 16, jnp.float32)
                        hi = plsc.bitcast(
                            wrd & jnp.uint32(0xFFFF0000), jnp.float32)
                        msg[i, pl.ds((2 * g) * chan, chan)] = (
                            lo * _lane_bcast(w, 2 * g))
                        msg[i, pl.ds((2 * g + 1) * chan, chan)] = (
                            hi * _lane_bcast(w, 2 * g + 1))
                else:
                    for h in range(heads):
                        wh = _lane_bcast(w, h)
                        msg[i, pl.ds(h * chan, chan)] = (
                            hs[i, pl.ds(h * chan, chan)] * wh)

        def _zrow(i, _):
            zero = jnp.zeros((16,), jnp.float32)
            for j in range(D // 16):
                msg0[i, pl.ds(j * 16, 16)] = zero
            w0[i, pl.ds(0, 16)] = zero
            return 0
        lax.fori_loop(0, B, _zrow, 0)
        rb = s * ROWS_PER_TILE
        full, rem = ROWS_PER_TILE // B, ROWS_PER_TILE % B
        for k in range(full):
            pltpu.sync_copy(msg0, num_s.at[pl.ds(rb + k * B, B)])
            pltpu.sync_copy(w0, den_s.at[pl.ds(rb + k * B, B)])
        if rem:
            pltpu.sync_copy(msg0.at[pl.ds(0, rem)],
                            num_s.at[pl.ds(rb + full * B, rem)])
            pltpu.sync_copy(w0.at[pl.ds(0, rem)],
                            den_s.at[pl.ds(rb + full * B, rem)])
        prime(slots[0], 0)
        prime(slots[1], 1)
        plsc.subcore_barrier()

        def _pair(p, _):
            b0 = 2 * p
            wait_gathers(slots[0])
            compute(slots[0])
            scatter(slots[0])
            wait_gathers(slots[1])
            compute(slots[1])
            scatter(slots[1])

            @pl.when(p < my_pairs - 1)
            def _():
                wait_scatter(slots[0])
                prime(slots[0], b0 + 2)
                wait_scatter(slots[1])
                prime(slots[1], b0 + 3)
            return 0
        lax.fori_loop(0, my_pairs, _pair, 0)
        wait_scatter(slots[0])
        wait_scatter(slots[1])

        plsc.subcore_barrier()
        pltpu.sync_copy(num_s.at[pl.ds(rb, ROWS_PER_TILE)],
                        num_hbm.at[c, pl.ds(rb, ROWS_PER_TILE)])
        pltpu.sync_copy(den_s.at[pl.ds(rb, ROWS_PER_TILE)],
                        den_hbm.at[c, pl.ds(rb, ROWS_PER_TILE)])

    return sc_gat


_sc_gat_l1 = _make_sc_gat(HEADS * HID, HEADS, 64, 12800, 8192, packed=True)
_sc_gat_l2 = _make_sc_gat(NCLASS, 1, 128, CHUNK, CHUNK)


def _head_matrix(a):
    h, cch = a.shape
    rows = jnp.arange(h * cch)
    cols = rows // cch
    m = jnp.zeros((h * cch, 8), jnp.float32).at[rows, cols].set(a.reshape(-1))
    return jnp.concatenate([m, m], axis=1)


def kernel(x, edge_index, edge_attr, W1, a_src1, a_dst1, b1,
           W2, a_src2, a_dst2, b2):
    acs = _head_matrix(a_src1)
    acd = _head_matrix(a_dst1)
    a2s = jnp.tile(a_src2.reshape(NCLASS, 1), (1, 16))
    a2d = jnp.tile(a_dst2.reshape(NCLASS, 1), (1, 16))
    s_rows = jnp.arange(16)
    s_cols = jnp.arange(HEADS * HID)
    s_mat = (s_rows[:, None] == (s_cols[None, :] // HID)).astype(jnp.float32)

    loop = jnp.arange(N, dtype=jnp.int32)
    pad = jnp.full((E_PAD - E - N,), N, dtype=jnp.int32)
    src_all = jnp.concatenate([edge_index[0], loop, pad])
    dst_all = jnp.concatenate([edge_index[1], loop, pad])

    h1, as1, ad1 = pl.pallas_call(
        _tc1_body,
        out_shape=[
            jax.ShapeDtypeStruct((NPAD, HEADS * HID), jnp.float32),
            jax.ShapeDtypeStruct((NPAD, 16), jnp.float32),
            jax.ShapeDtypeStruct((NPAD, 16), jnp.float32),
        ],
    )(x, W1, acs, acd)

    u = lax.bitcast_convert_type(h1.astype(jnp.bfloat16),
                                 jnp.uint16).astype(jnp.uint32)
    ur = u.reshape(NPAD, 4, 2, 16)
    h1w = (ur[:, :, 0, :] | (ur[:, :, 1, :] << 16)).reshape(NPAD, 64)

    num1, den1 = _sc_gat_l1(src_all, dst_all, h1w, as1, ad1)

    h2, as2, ad2 = pl.pallas_call(
        _tc2_body,
        out_shape=[
            jax.ShapeDtypeStruct((NPAD, NCLASS), jnp.float32),
            jax.ShapeDtypeStruct((NPAD, 16), jnp.float32),
            jax.ShapeDtypeStruct((NPAD, 16), jnp.float32),
        ],
    )(num1, den1, s_mat, b1.reshape(1, HEADS * HID), W2, a2s, a2d)

    num2, den2 = _sc_gat_l2(src_all, dst_all, h2, as2, ad2)

    final, logp = pl.pallas_call(
        _tc3_body,
        out_shape=[
            jax.ShapeDtypeStruct((N, NCLASS), jnp.float32),
            jax.ShapeDtypeStruct((N, NCLASS), jnp.float32),
        ],
    )(num2, den2, b2.reshape(1, NCLASS))

    return (final, logp)


# speedup vs baseline: 84.5823x; 1.0176x over previous
import functools

import jax
import jax.numpy as jnp
from jax import lax
from jax.experimental import pallas as pl
from jax.experimental.pallas import tpu as pltpu
from jax.experimental.pallas import tpu_sc as plsc

N = 10000
NFEAT = 128
HID = 16
HEADS = 8
NCLASS = 16
E = 320000

NW = 32
CHUNK = 10496
E_PAD = NW * CHUNK
NPAD = 10112
ROWS_PER_TILE = NPAD // 16

_NEG = -1.0e30

_GATHER_DN = lax.GatherDimensionNumbers(
    offset_dims=(), collapsed_slice_dims=(0,), start_index_map=(0,))


def _lane_bcast(v, h):
    idx = jnp.full((16, 1), h, dtype=jnp.int32)
    return lax.gather(v, idx, _GATHER_DN, (1,),
                      mode=lax.GatherScatterMode.PROMISE_IN_BOUNDS)


def _pad_rows(a, value):
    return jnp.pad(a, ((0, NPAD - N), (0, 0)), constant_values=value)


def _tc1_body(x_ref, w1_ref, acs_ref, acd_ref, h_ref, as_ref, ad_ref):
    h = jnp.dot(x_ref[...], w1_ref[...], preferred_element_type=jnp.float32)
    h_ref[...] = _pad_rows(h, 0.0)
    as_ref[...] = _pad_rows(
        jnp.dot(h, acs_ref[...], preferred_element_type=jnp.float32), _NEG)
    ad_ref[...] = _pad_rows(
        jnp.dot(h, acd_ref[...], preferred_element_type=jnp.float32), _NEG)


def _tc2_body(num_ref, den_ref, s_ref, b1_ref, w2_ref,
              a2s_ref, a2d_ref, h2_ref, as2_ref, ad2_ref):
    den = den_ref[0, :N] + den_ref[1, :N]
    div = jnp.dot(den, s_ref[...], preferred_element_type=jnp.float32)
    out1 = (num_ref[0, :N] + num_ref[1, :N]) / div + b1_ref[...]
    x2 = jnp.where(out1 > 0, out1, jnp.exp(out1) - 1.0)
    h2 = jnp.dot(x2, w2_ref[...], preferred_element_type=jnp.float32)
    h2_ref[...] = _pad_rows(h2, 0.0)
    as2_ref[...] = _pad_rows(
        jnp.dot(h2, a2s_ref[...], preferred_element_type=jnp.float32), _NEG)
    ad2_ref[...] = _pad_rows(
        jnp.dot(h2, a2d_ref[...], preferred_element_type=jnp.float32), _NEG)


def _tc3_body(num_ref, den_ref, b2_ref, fin_ref, lp_ref):
    fin = ((num_ref[0, :N] + num_ref[1, :N])
           / (den_ref[0, :N] + den_ref[1, :N]) + b2_ref[...])
    fin_ref[...] = fin
    m = jnp.max(fin, axis=1, keepdims=True)
    lse = jnp.log(jnp.sum(jnp.exp(fin - m), axis=1, keepdims=True)) + m
    lp_ref[...] = fin - lse


def _make_sc_gat(D, heads, B, chunk0, chunk1, packed=False):
    mesh = plsc.VectorSubcoreMesh(core_axis_name="c", subcore_axis_name="s")
    chan = D // heads
    assert chunk0 + chunk1 == 2 * CHUNK
    nb0, nb1 = chunk0 // B, chunk1 // B
    assert nb0 * B == chunk0 and nb1 * B == chunk1
    assert nb0 % 2 == 0 and nb1 % 2 == 0
    pairs0, pairs1 = nb0 // 2, nb1 // 2
    off1 = 16 * chunk0

    @functools.partial(
        pl.kernel,
        out_type=[
            jax.ShapeDtypeStruct((2, NPAD, D), jnp.float32),
            jax.ShapeDtypeStruct((2, NPAD, 16), jnp.float32),
        ],
        mesh=mesh,
        compiler_params=pltpu.CompilerParams(use_tc_tiling_on_sc=False, needs_layout_passes=False),
        scratch_types=[
            pltpu.VMEM((B,), jnp.int32),
            pltpu.VMEM((B,), jnp.int32),
            (pltpu.VMEM((B, D // 2), jnp.uint32) if packed
             else pltpu.VMEM((B, D), jnp.float32)),
            pltpu.VMEM((B, D), jnp.float32),
            pltpu.VMEM((B, 16), jnp.float32),
            pltpu.VMEM((B, 16), jnp.float32),
            pltpu.VMEM((B, 16), jnp.float32),
            pltpu.VMEM((B,), jnp.int32),
            pltpu.VMEM((B,), jnp.int32),
            (pltpu.VMEM((B, D // 2), jnp.uint32) if packed
             else pltpu.VMEM((B, D), jnp.float32)),
            pltpu.VMEM((B, D), jnp.float32),
            pltpu.VMEM((B, 16), jnp.float32),
            pltpu.VMEM((B, 16), jnp.float32),
            pltpu.VMEM((B, 16), jnp.float32),
            pltpu.VMEM_SHARED((NPAD, D), jnp.float32),
            pltpu.VMEM_SHARED((NPAD, 16), jnp.float32),
            pltpu.SemaphoreType.DMA,
            pltpu.SemaphoreType.DMA,
            pltpu.SemaphoreType.DMA,
            pltpu.SemaphoreType.DMA,
        ],
    )
    def sc_gat(src_hbm, dst_hbm, h_hbm, as_hbm, ad_hbm, num_hbm, den_hbm,
               src0, dst0, hs0, msg0, as0, ad0, w0,
               src1, dst1, hs1, msg1, as1, ad1, w1,
               num_s, den_s, sg0, sg1, ss0, ss1):
        c = lax.axis_index("c")
        s = lax.axis_index("s")
        tile_base = jnp.where(c == 0, s * chunk0, off1 + s * chunk1)
        my_pairs = jnp.where(c == 0, pairs0, pairs1)
        slots = ((src0, dst0, hs0, msg0, as0, ad0, w0, sg0, ss0),
                 (src1, dst1, hs1, msg1, as1, ad1, w1, sg1, ss1))

        def prime(S, b):
            sv, dv, hs, msg, asv, adv, wv, sg, ss = S
            base = tile_base + b * B
            pltpu.sync_copy(src_hbm.at[pl.ds(base, B)], sv)
            pltpu.sync_copy(dst_hbm.at[pl.ds(base, B)], dv)
            pltpu.async_copy(h_hbm.at[sv], hs, sg)
            pltpu.async_copy(as_hbm.at[sv], asv, sg)
            pltpu.async_copy(ad_hbm.at[dv], adv, sg)

        def wait_gathers(S):
            sv, dv, hs, msg, asv, adv, wv, sg, ss = S
            pltpu.make_async_copy(h_hbm.at[sv], hs, sg).wait()
            pltpu.make_async_copy(as_hbm.at[sv], asv, sg).wait()
            pltpu.make_async_copy(ad_hbm.at[dv], adv, sg).wait()

        def scatter(S):
            sv, dv, hs, msg, asv, adv, wv, sg, ss = S
            pltpu.async_copy(msg, num_s.at[dv], ss, add=True)
            pltpu.async_copy(wv, den_s.at[dv], ss, add=True)

        def wait_scatter(S):
            sv, dv, hs, msg, asv, adv, wv, sg, ss = S
            pltpu.make_async_copy(msg, num_s.at[dv], ss).wait()
            pltpu.make_async_copy(wv, den_s.at[dv], ss).wait()

        def compute(S):
            sv, dv, hs, msg, asv, adv, wv, sg, ss = S

            @plsc.parallel_loop(0, B, 1, unroll=8)
            def _edge(i):
                e = asv[i] + adv[i]
                e = jnp.maximum(e, 0.2 * e)
                w = jnp.exp(e)
                wv[i] = w
                if heads == 1:
                    msg[i] = hs[i] * w
                elif packed:
                    for g in range(heads // 2):
                        wrd = hs[i, pl.ds(g * chan, chan)]
                        lo = plsc.bitcast(wrd << 16, jnp.float32)
                        hi = plsc.bitcast(
                            wrd & jnp.uint32(0xFFFF0000), jnp.float32)
                        msg[i, pl.ds((2 * g) * chan, chan)] = (
                            lo * _lane_bcast(w, 2 * g))
                        msg[i, pl.ds((2 * g + 1) * chan, chan)] = (
                            hi * _lane_bcast(w, 2 * g + 1))
                else:
                    for h in range(heads):
                        wh = _lane_bcast(w, h)
                        msg[i, pl.ds(h * chan, chan)] = (
                            hs[i, pl.ds(h * chan, chan)] * wh)

        def _zrow(i, _):
            zero = jnp.zeros((16,), jnp.float32)
            for j in range(D // 16):
                msg0[i, pl.ds(j * 16, 16)] = zero
            w0[i, pl.ds(0, 16)] = zero
            return 0
        lax.fori_loop(0, B, _zrow, 0)
        rb = s * ROWS_PER_TILE
        full, rem = ROWS_PER_TILE // B, ROWS_PER_TILE % B
        for k in range(full):
            pltpu.sync_copy(msg0, num_s.at[pl.ds(rb + k * B, B)])
            pltpu.sync_copy(w0, den_s.at[pl.ds(rb + k * B, B)])
        if rem:
            pltpu.sync_copy(msg0.at[pl.ds(0, rem)],
                            num_s.at[pl.ds(rb + full * B, rem)])
            pltpu.sync_copy(w0.at[pl.ds(0, rem)],
                            den_s.at[pl.ds(rb + full * B, rem)])
        prime(slots[0], 0)
        prime(slots[1], 1)
        plsc.subcore_barrier()

        def _pair(p, _):
            b0 = 2 * p
            wait_gathers(slots[0])
            compute(slots[0])
            scatter(slots[0])
            wait_gathers(slots[1])
            compute(slots[1])
            scatter(slots[1])

            @pl.when(p < my_pairs - 1)
            def _():
                wait_scatter(slots[0])
                prime(slots[0], b0 + 2)
                wait_scatter(slots[1])
                prime(slots[1], b0 + 3)
            return 0
        lax.fori_loop(0, my_pairs, _pair, 0)
        wait_scatter(slots[0])
        wait_scatter(slots[1])

        plsc.subcore_barrier()
        pltpu.sync_copy(num_s.at[pl.ds(rb, ROWS_PER_TILE)],
                        num_hbm.at[c, pl.ds(rb, ROWS_PER_TILE)])
        pltpu.sync_copy(den_s.at[pl.ds(rb, ROWS_PER_TILE)],
                        den_hbm.at[c, pl.ds(rb, ROWS_PER_TILE)])

    return sc_gat


_sc_gat_l1 = _make_sc_gat(HEADS * HID, HEADS, 64, 12672, 8320, packed=True)
_sc_gat_l2 = _make_sc_gat(NCLASS, 1, 128, 10752, 10240)


def _head_matrix(a):
    h, cch = a.shape
    rows = jnp.arange(h * cch)
    cols = rows // cch
    m = jnp.zeros((h * cch, 8), jnp.float32).at[rows, cols].set(a.reshape(-1))
    return jnp.concatenate([m, m], axis=1)


def kernel(x, edge_index, edge_attr, W1, a_src1, a_dst1, b1,
           W2, a_src2, a_dst2, b2):
    acs = _head_matrix(a_src1)
    acd = _head_matrix(a_dst1)
    a2s = jnp.tile(a_src2.reshape(NCLASS, 1), (1, 16))
    a2d = jnp.tile(a_dst2.reshape(NCLASS, 1), (1, 16))
    s_rows = jnp.arange(16)
    s_cols = jnp.arange(HEADS * HID)
    s_mat = (s_rows[:, None] == (s_cols[None, :] // HID)).astype(jnp.float32)

    loop = jnp.arange(N, dtype=jnp.int32)
    pad = jnp.full((E_PAD - E - N,), N, dtype=jnp.int32)
    src_all = jnp.concatenate([edge_index[0], loop, pad])
    dst_all = jnp.concatenate([edge_index[1], loop, pad])

    h1, as1, ad1 = pl.pallas_call(
        _tc1_body,
        out_shape=[
            jax.ShapeDtypeStruct((NPAD, HEADS * HID), jnp.float32),
            jax.ShapeDtypeStruct((NPAD, 16), jnp.float32),
            jax.ShapeDtypeStruct((NPAD, 16), jnp.float32),
        ],
    )(x, W1, acs, acd)

    u = lax.bitcast_convert_type(h1.astype(jnp.bfloat16),
                                 jnp.uint16).astype(jnp.uint32)
    ur = u.reshape(NPAD, 4, 2, 16)
    h1w = (ur[:, :, 0, :] | (ur[:, :, 1, :] << 16)).reshape(NPAD, 64)

    num1, den1 = _sc_gat_l1(src_all, dst_all, h1w, as1, ad1)

    h2, as2, ad2 = pl.pallas_call(
        _tc2_body,
        out_shape=[
            jax.ShapeDtypeStruct((NPAD, NCLASS), jnp.float32),
            jax.ShapeDtypeStruct((NPAD, 16), jnp.float32),
            jax.ShapeDtypeStruct((NPAD, 16), jnp.float32),
        ],
    )(num1, den1, s_mat, b1.reshape(1, HEADS * HID), W2, a2s, a2d)

    num2, den2 = _sc_gat_l2(src_all, dst_all, h2, as2, ad2)

    final, logp = pl.pallas_call(
        _tc3_body,
        out_shape=[
            jax.ShapeDtypeStruct((N, NCLASS), jnp.float32),
            jax.ShapeDtypeStruct((N, NCLASS), jnp.float32),
        ],
    )(num2, den2, b2.reshape(1, NCLASS))

    return (final, logp)
